# baseline (speedup 1.0000x reference)
import functools

import jax
import jax.numpy as jnp
from jax import lax
from jax.experimental import pallas as pl
from jax.experimental.pallas import tpu as pltpu
from jax.experimental.pallas import tpu_sc as plsc

EMBED = 64

_NC = 2
_NS = 16
_NW = _NC * _NS

_BCHUNK = 8


def _make_format(vocab):
    nbf = vocab // 128
    tail = vocab - nbf * 128
    trips = (nbf + _NW - 1) // _NW
    half_trips = (trips + 1) // 2
    tail_wid = nbf % _NW
    mesh = plsc.VectorSubcoreMesh(core_axis_name="c", subcore_axis_name="s")

    @functools.partial(
        pl.kernel,
        mesh=mesh,
        out_type=jax.ShapeDtypeStruct((vocab * EMBED,), jnp.float32),
        scratch_types=[
            pltpu.VMEM((EMBED, 129), jnp.float32),
            pltpu.VMEM((EMBED, 129), jnp.float32),
            pltpu.VMEM((128 * EMBED,), jnp.float32),
            pltpu.VMEM((128 * EMBED,), jnp.float32),
            pltpu.VMEM((EMBED, tail), jnp.float32),
            pltpu.VMEM((tail * EMBED,), jnp.float32),
            pltpu.SemaphoreType.DMA,
            pltpu.SemaphoreType.DMA,
            pltpu.SemaphoreType.DMA,
            pltpu.SemaphoreType.DMA,
        ],
        compiler_params=pltpu.CompilerParams(needs_layout_passes=False, disable_bounds_checks=True),
    )
    def fmt(tt_hbm, s_hbm, src0, src1, dst0, dst1, srct, dstt,
            si0, si1, so0, so1):
        wid = lax.axis_index("s") * _NC + lax.axis_index("c")

        lane = lax.iota(jnp.int32, 16)
        erows = [lane + 16 * c for c in range(EMBED // 16)]

        def load(b, src, sem):
            v0 = pl.multiple_of(b * 128, 128)
            pltpu.async_copy(
                tt_hbm.at[:, pl.ds(v0, 128)], src.at[:, pl.ds(0, 128)], sem)

        def wait_load(src, sem):
            pltpu.make_async_copy(
                tt_hbm.at[:, pl.ds(0, 128)], src.at[:, pl.ds(0, 128)],
                sem).wait()

        def store(b, dst, sem):
            f0 = pl.multiple_of(b * (128 * EMBED), 128 * EMBED)
            pltpu.async_copy(dst, s_hbm.at[pl.ds(f0, 128 * EMBED)], sem)

        def wait_store(dst, sem):
            pltpu.make_async_copy(
                dst, s_hbm.at[pl.ds(0, 128 * EMBED)], sem).wait()

        def transpose(src, dst, ntok):
            @plsc.parallel_loop(0, ntok, unroll=4)
            def vcol(v):
                col = jnp.full((16,), v, jnp.int32)
                for c in range(EMBED // 16):
                    vals = plsc.load_gather(src, [erows[c], col])
                    dst[pl.ds(v * EMBED + c * 16, 16)] = vals

        def b_of(t):
            return t * _NW + wid

        load(b_of(0), src0, si0)

        def body(u, carry):
            b0 = b_of(2 * u)
            b1 = b_of(2 * u + 1)
            b2 = b_of(2 * u + 2)

            @pl.when(b1 < nbf)
            def _():
                load(b1, src1, si1)

            @pl.when(b0 < nbf)
            def _():
                wait_load(src0, si0)

                @pl.when(u > 0)
                def _():
                    wait_store(dst0, so0)

                transpose(src0, dst0, 128)
                store(b0, dst0, so0)

            @pl.when(b2 < nbf)
            def _():
                load(b2, src0, si0)

            @pl.when(b1 < nbf)
            def _():
                wait_load(src1, si1)

                @pl.when(u > 0)
                def _():
                    wait_store(dst1, so1)

                transpose(src1, dst1, 128)
                store(b1, dst1, so1)

            return carry

        lax.fori_loop(0, half_trips, body, 0)
        wait_store(dst0, so0)
        wait_store(dst1, so1)

        @pl.when(wid == tail_wid)
        def _():
            v0 = nbf * 128
            pltpu.sync_copy(tt_hbm.at[:, pl.ds(v0, tail)], srct)

            @plsc.parallel_loop(0, tail, unroll=4)
            def vcol(v):
                col = jnp.full((16,), v, jnp.int32)
                for c in range(EMBED // 16):
                    vals = plsc.load_gather(srct, [erows[c], col])
                    dstt[pl.ds(v * EMBED + c * 16, 16)] = vals
            pltpu.sync_copy(
                dstt, s_hbm.at[pl.ds(v0 * EMBED, tail * EMBED)])

    return fmt


def _make_gather(nb, nl):
    assert nb % (_NW * 2 * _BCHUNK) == 0
    bpw = nb // _NW
    nchunk = bpw // _BCHUNK
    ntrips = nchunk // 2
    mesh = plsc.VectorSubcoreMesh(core_axis_name="c", subcore_axis_name="s")

    @functools.partial(
        pl.kernel,
        mesh=mesh,
        out_type=jax.ShapeDtypeStruct((nb, nl, EMBED), jnp.float32),
        scratch_types=[
            pltpu.VMEM((bpw, nl), jnp.int32),
            pltpu.VMEM((_BCHUNK, nl, EMBED), jnp.float32),
            pltpu.VMEM((_BCHUNK, nl, EMBED), jnp.float32),
            pltpu.SemaphoreType.DMA,
            pltpu.SemaphoreType.DMA,
        ],
        compiler_params=pltpu.CompilerParams(use_tc_tiling_on_sc=False, disable_bounds_checks=True),
    )
    def k(idx_hbm, table_hbm, out_hbm, idx_all, rows0, rows1, sem0, sem1):
        wid = lax.axis_index("s") * _NC + lax.axis_index("c")
        base = wid * bpw

        pltpu.sync_copy(idx_hbm.at[pl.ds(base, bpw)], idx_all)

        def issue(c, rows, sem):
            for j in range(_BCHUNK):
                pltpu.async_copy(
                    table_hbm.at[idx_all.at[c * _BCHUNK + j]],
                    rows.at[j],
                    sem,
                )

        def drain(c, rows, sem):
            for j in range(_BCHUNK):
                pltpu.make_async_copy(
                    table_hbm.at[idx_all.at[c * _BCHUNK + j]],
                    rows.at[j],
                    sem,
                ).wait()

        def out_copy(c, rows):
            start = pl.multiple_of(base + c * _BCHUNK, _BCHUNK)
            pltpu.sync_copy(rows, out_hbm.at[pl.ds(start, _BCHUNK)])

        issue(0, rows0, sem0)

        def body(t, carry):
            c0 = 2 * t
            issue(c0 + 1, rows1, sem1)
            drain(c0, rows0, sem0)
            out_copy(c0, rows0)
            issue(c0 + 2, rows0, sem0)
            drain(c0 + 1, rows1, sem1)
            out_copy(c0 + 1, rows1)
            return carry

        lax.fori_loop(0, ntrips - 1, body, 0)

        c0 = nchunk - 2
        issue(c0 + 1, rows1, sem1)
        drain(c0, rows0, sem0)
        out_copy(c0, rows0)
        drain(c0 + 1, rows1, sem1)
        out_copy(c0 + 1, rows1)

    return k


def kernel(input, table):
    b, l = input.shape
    v, d = table.shape
    s = _make_format(v)(table.T)
    return _make_gather(b, l)(input.astype(jnp.int32), s.reshape(v, d))


# speedup vs baseline: 1.1509x; 1.1509x over previous
import functools

import jax
import jax.numpy as jnp
from jax import lax
from jax.experimental import pallas as pl
from jax.experimental.pallas import tpu as pltpu
from jax.experimental.pallas import tpu_sc as plsc

EMBED = 64

_NC = 2
_NS = 16
_NW = _NC * _NS

_BCHUNK = 8


def _make_gather(nb, nl):
    assert nb % (_NW * 2 * _BCHUNK) == 0
    bpw = nb // _NW
    nchunk = bpw // _BCHUNK
    ntrips = nchunk // 2
    mesh = plsc.VectorSubcoreMesh(core_axis_name="c", subcore_axis_name="s")

    @functools.partial(
        pl.kernel,
        mesh=mesh,
        out_type=jax.ShapeDtypeStruct((nb, nl, EMBED), jnp.float32),
        scratch_types=[
            pltpu.VMEM((bpw, nl), jnp.int32),
            pltpu.VMEM((_BCHUNK, nl, EMBED), jnp.float32),
            pltpu.VMEM((_BCHUNK, nl, EMBED), jnp.float32),
            pltpu.SemaphoreType.DMA,
            pltpu.SemaphoreType.DMA,
        ],
        compiler_params=pltpu.CompilerParams(use_tc_tiling_on_sc=False, disable_bounds_checks=True),
    )
    def k(idx_hbm, table_hbm, out_hbm, idx_all, rows0, rows1, sem0, sem1):
        wid = lax.axis_index("s") * _NC + lax.axis_index("c")
        base = wid * bpw

        pltpu.sync_copy(idx_hbm.at[pl.ds(base, bpw)], idx_all)

        def issue(c, rows, sem):
            for j in range(_BCHUNK):
                pltpu.async_copy(
                    table_hbm.at[idx_all.at[c * _BCHUNK + j]],
                    rows.at[j],
                    sem,
                )

        def drain(c, rows, sem):
            for j in range(_BCHUNK):
                pltpu.make_async_copy(
                    table_hbm.at[idx_all.at[c * _BCHUNK + j]],
                    rows.at[j],
                    sem,
                ).wait()

        def out_copy(c, rows):
            start = pl.multiple_of(base + c * _BCHUNK, _BCHUNK)
            pltpu.sync_copy(rows, out_hbm.at[pl.ds(start, _BCHUNK)])

        issue(0, rows0, sem0)

        def body(t, carry):
            c0 = 2 * t
            issue(c0 + 1, rows1, sem1)
            drain(c0, rows0, sem0)
            out_copy(c0, rows0)
            issue(c0 + 2, rows0, sem0)
            drain(c0 + 1, rows1, sem1)
            out_copy(c0 + 1, rows1)
            return carry

        lax.fori_loop(0, ntrips - 1, body, 0)

        c0 = nchunk - 2
        issue(c0 + 1, rows1, sem1)
        drain(c0, rows0, sem0)
        out_copy(c0, rows0)
        drain(c0 + 1, rows1, sem1)
        out_copy(c0 + 1, rows1)

    return k


def kernel(input, table):
    b, l = input.shape
    return _make_gather(b, l)(input.astype(jnp.int32), table)
